# baseline (speedup 1.0000x reference)
import functools

import jax
import jax.numpy as jnp
from jax import lax
from jax.experimental import pallas as pl
from jax.experimental.pallas import tpu as pltpu
from jax.experimental.pallas import tpu_sc as plsc

N_VOCAB = 123
VPAD = 128
B, L = 4096, 20
NTOK = B * L
NC, NS = 2, 16
NW = NC * NS
CHUNK = 128
TOK_PER_W = NTOK // NW
NCHUNK = TOK_PER_W // CHUNK


def _table_body(emb_ref, w1_ref, b1_ref, w2_ref, b2_ref, out_ref):
    x = jnp.maximum(emb_ref[...], 0.0)
    h = jnp.dot(x, w1_ref[...], preferred_element_type=jnp.float32)
    h = jnp.maximum(h + b1_ref[...], 0.0)
    t = jnp.dot(h, w2_ref[...], preferred_element_type=jnp.float32)
    out_ref[...] = t + b2_ref[...]


def _compute_table(embed, W1, b1, W2, b2):
    w2_pad = jnp.pad(W2, ((0, 0), (0, VPAD - N_VOCAB)))
    b2_pad = jnp.pad(b2, (0, VPAD - N_VOCAB)).reshape(1, VPAD)
    return pl.pallas_call(
        _table_body,
        out_shape=jax.ShapeDtypeStruct((N_VOCAB, VPAD), jnp.float32),
    )(embed, W1, b1.reshape(1, -1), w2_pad, b2_pad)


def _gather_body(table_hbm, idx_hbm, out_hbm, idx_v, rows, gsems, wsems):
    c = lax.axis_index("c")
    s = lax.axis_index("s")
    wid = s * NC + c
    base = wid * TOK_PER_W

    pltpu.sync_copy(idx_hbm.at[wid], idx_v)
    grabs = [None, None]
    writes = [None, None]
    grabs[0] = pltpu.async_copy(table_hbm.at[idx_v.at[0]], rows[0], gsems[0])
    for j in range(NCHUNK):
        b = j % 2
        if j + 1 < NCHUNK:
            if writes[b ^ 1] is not None:
                writes[b ^ 1].wait()
            grabs[b ^ 1] = pltpu.async_copy(
                table_hbm.at[idx_v.at[j + 1]], rows[b ^ 1], gsems[b ^ 1]
            )
        grabs[b].wait()
        writes[b] = pltpu.async_copy(
            rows[b], out_hbm.at[pl.ds(base + j * CHUNK, CHUNK)], wsems[b]
        )
    writes[0].wait()
    writes[1].wait()


_gather = functools.partial(
    pl.kernel,
    out_type=jax.ShapeDtypeStruct((NTOK, VPAD), jnp.float32),
    mesh=plsc.VectorSubcoreMesh(
        core_axis_name="c", subcore_axis_name="s", num_cores=NC, num_subcores=NS
    ),
    scratch_types=[
        pltpu.VMEM((NCHUNK, CHUNK), jnp.int32),
        [pltpu.VMEM((CHUNK, VPAD), jnp.float32) for _ in range(2)],
        [pltpu.SemaphoreType.DMA for _ in range(2)],
        [pltpu.SemaphoreType.DMA for _ in range(2)],
    ],
)(_gather_body)


def kernel(inputs, embed, W1, b1, W2, b2):
    table = _compute_table(embed, W1, b1, W2, b2)
    idx = inputs.reshape(-1).astype(jnp.int32).reshape(NW, NCHUNK, CHUNK)
    out = _gather(table, idx)
    return out[:, :N_VOCAB].reshape(B, L, N_VOCAB)


# speedup vs baseline: 1.2006x; 1.2006x over previous
import functools

import jax
import jax.numpy as jnp
from jax import lax
from jax.experimental import pallas as pl
from jax.experimental.pallas import tpu as pltpu
from jax.experimental.pallas import tpu_sc as plsc

N_VOCAB = 123
VPAD = 128
B, L = 4096, 20
NTOK = B * L
NC, NS = 2, 16
NW = NC * NS
CHUNK = 128
TOK_PER_W = NTOK // NW
NCHUNK = TOK_PER_W // CHUNK
GROUP = 4


def _table_body(emb_ref, w1_ref, b1_ref, w2_ref, b2_ref, out_ref):
    x = jnp.maximum(emb_ref[...], 0.0)
    h = jnp.dot(x, w1_ref[...], preferred_element_type=jnp.float32)
    h = jnp.maximum(h + b1_ref[...], 0.0)
    t = jnp.dot(h, w2_ref[...], preferred_element_type=jnp.float32)
    out_ref[...] = t + b2_ref[...]


def _compute_table(embed, W1, b1, W2, b2):
    return pl.pallas_call(
        _table_body,
        out_shape=jax.ShapeDtypeStruct((N_VOCAB, N_VOCAB), jnp.float32),
    )(embed, W1, b1.reshape(1, -1), W2, b2.reshape(1, -1))


def _gather_body(table_hbm, idx_hbm, out_hbm, idx_v, rows_v, sem):
    c = lax.axis_index("c")
    s = lax.axis_index("s")
    wid = s * NC + c
    base = wid * TOK_PER_W

    pltpu.sync_copy(idx_hbm.at[wid], idx_v)
    for g in range(NCHUNK // GROUP):
        grabs = [
            pltpu.async_copy(
                table_hbm.at[idx_v.at[g * GROUP + k]],
                rows_v.at[pl.ds(k * CHUNK, CHUNK)],
                sem,
            )
            for k in range(GROUP)
        ]
        for d in grabs:
            d.wait()
        pltpu.sync_copy(
            rows_v, out_hbm.at[pl.ds(base + g * GROUP * CHUNK, GROUP * CHUNK)]
        )


_gather = functools.partial(
    pl.kernel,
    out_type=jax.ShapeDtypeStruct((NTOK, N_VOCAB), jnp.float32),
    mesh=plsc.VectorSubcoreMesh(
        core_axis_name="c", subcore_axis_name="s", num_cores=NC, num_subcores=NS
    ),
    scratch_types=[
        pltpu.VMEM((NCHUNK, CHUNK), jnp.int32),
        pltpu.VMEM((GROUP * CHUNK, N_VOCAB), jnp.float32),
        pltpu.SemaphoreType.DMA,
    ],
    compiler_params=pltpu.CompilerParams(use_tc_tiling_on_sc=False),
)(_gather_body)


def kernel(inputs, embed, W1, b1, W2, b2):
    table = _compute_table(embed, W1, b1, W2, b2)
    idx = inputs.reshape(-1).astype(jnp.int32).reshape(NW, NCHUNK, CHUNK)
    out = _gather(table, idx)
    return out.reshape(B, L, N_VOCAB)


# speedup vs baseline: 1.2007x; 1.0001x over previous
import functools

import jax
import jax.numpy as jnp
from jax import lax
from jax.experimental import pallas as pl
from jax.experimental.pallas import tpu as pltpu
from jax.experimental.pallas import tpu_sc as plsc

N_VOCAB = 123
VPAD = 128
B, L = 4096, 20
NTOK = B * L
NC, NS = 2, 16
NW = NC * NS
CHUNK = 128
TOK_PER_W = NTOK // NW
NCHUNK = TOK_PER_W // CHUNK
GROUP = 4


def _table_body(emb_ref, w1_ref, b1_ref, w2_ref, b2_ref, out_ref):
    x = jnp.maximum(emb_ref[...], 0.0)
    h = jnp.dot(x, w1_ref[...], preferred_element_type=jnp.float32)
    h = jnp.maximum(h + b1_ref[...], 0.0)
    t = jnp.dot(h, w2_ref[...], preferred_element_type=jnp.float32)
    out_ref[...] = t + b2_ref[...]


def _compute_table(embed, W1, b1, W2, b2):
    return pl.pallas_call(
        _table_body,
        out_shape=jax.ShapeDtypeStruct((N_VOCAB, N_VOCAB), jnp.float32),
    )(embed, W1, b1.reshape(1, -1), W2, b2.reshape(1, -1))


def _gather_body(table_hbm, idx_hbm, out_hbm, idx_v, rows_v, sem):
    c = lax.axis_index("c")
    s = lax.axis_index("s")
    wid = s * NC + c
    base = wid * TOK_PER_W

    pltpu.sync_copy(idx_hbm.at[wid], idx_v)
    for g in range(NCHUNK // GROUP):
        for k in range(GROUP):
            pltpu.async_copy(
                table_hbm.at[idx_v.at[g * GROUP + k]],
                rows_v.at[pl.ds(k * CHUNK, CHUNK)],
                sem,
            ).wait()
        pltpu.sync_copy(
            rows_v, out_hbm.at[pl.ds(base + g * GROUP * CHUNK, GROUP * CHUNK)]
        )


_gather = functools.partial(
    pl.kernel,
    out_type=jax.ShapeDtypeStruct((NTOK, N_VOCAB), jnp.float32),
    mesh=plsc.VectorSubcoreMesh(
        core_axis_name="c", subcore_axis_name="s", num_cores=NC, num_subcores=NS
    ),
    scratch_types=[
        pltpu.VMEM((NCHUNK, CHUNK), jnp.int32),
        pltpu.VMEM((GROUP * CHUNK, N_VOCAB), jnp.float32),
        pltpu.SemaphoreType.DMA,
    ],
    compiler_params=pltpu.CompilerParams(use_tc_tiling_on_sc=False),
)(_gather_body)


def kernel(inputs, embed, W1, b1, W2, b2):
    table = _compute_table(embed, W1, b1, W2, b2)
    idx = inputs.reshape(-1).astype(jnp.int32).reshape(NW, NCHUNK, CHUNK)
    out = _gather(table, idx)
    return out.reshape(B, L, N_VOCAB)
